# Initial kernel scaffold; baseline (speedup 1.0000x reference)
import jax
import jax.numpy as jnp
from jax.experimental import pallas as pl


def kernel(token_ids, emb, W_ih_f, W_hh_f, b_ih_f, b_hh_f, W_ih_b, W_hh_b, b_ih_b, b_hh_b, fc_W, fc_b):
    raise NotImplementedError("write your pallas kernel here")


# speedup vs baseline: 9.2336x; 9.2336x over previous
import jax
import jax.numpy as jnp
from jax.experimental import pallas as pl
from jax.experimental.pallas import tpu as pltpu

B, S, E, H, V, G = 32, 2048, 300, 256, 50000, 5
VT = 2000
N_VT = V // VT
GPAD = 128


def _proj_kernel(emb_ref, w_ref, b_ref, out_ref):
    res = jnp.dot(emb_ref[...], w_ref[0], preferred_element_type=jnp.float32)
    out_ref[0] = res + b_ref[0]


def _scan_kernel(idx_ref, p_hbm, whh_ref, fcw_ref, out_ref, p_vmem, tile, sem):
    d = pl.program_id(0)
    fwd = d == 0
    cp = pltpu.make_async_copy(p_hbm.at[pl.ds(d * (2 * V), 2 * V), :], p_vmem, sem)
    cp.start()
    cp.wait()

    whh = whh_ref[0]

    def gather(tstep, slot):
        base = tstep * B
        for mi in range(B):
            i = pl.multiple_of(idx_ref[base + mi], 2)
            tile.at[slot][mi:mi + 80:40, :] = p_vmem[pl.ds(i, 2), :]

    gather(jnp.where(fwd, 0, S - 1), 0)

    def step(t, carry):
        h, acc = carry
        slot = jax.lax.rem(t, 2)
        c0 = tile[slot, 0:B, :]
        c1 = tile[slot, 40:40 + B, :]
        xp = jnp.concatenate([c0, c1], axis=1)
        pre = jnp.dot(h, whh, preferred_element_type=jnp.float32) + xp
        tn = jnp.clip(t + 1, 0, S - 1)
        gather(jnp.where(fwd, tn, (S - 1) - tn), 1 - slot)
        h2 = jnp.tanh(pre)
        return h2, acc + h2

    zeros = jnp.zeros((B, H), jnp.float32)
    _, acc = jax.lax.fori_loop(0, S, step, (zeros, zeros))

    act = jnp.tanh(acc * (1.0 / S))
    out_ref[0] = jnp.dot(act, fcw_ref[0], preferred_element_type=jnp.float32)


def _impl(token_ids, emb, W_ih_f, W_hh_f, b_ih_f, b_hh_f,
          W_ih_b, W_hh_b, b_ih_b, b_hh_b, fc_W, fc_b, interpret=False):
    w_stack = jnp.stack([W_ih_f.T, W_ih_b.T])
    bias = jnp.stack([(b_ih_f + b_hh_f), (b_ih_b + b_hh_b)])
    bias = bias.reshape(2, 1, H)
    whh_stack = jnp.stack([W_hh_f.T, W_hh_b.T])
    fcwT = fc_W.T
    fcw_stack = jnp.stack([fcwT[:H], fcwT[H:]])
    fcw_stack = jnp.pad(fcw_stack, ((0, 0), (0, 0), (0, GPAD - G)))
    idx2 = (token_ids.T * 2).reshape(-1).astype(jnp.int32)

    proj = pl.pallas_call(
        _proj_kernel,
        grid=(2, N_VT),
        in_specs=[
            pl.BlockSpec((VT, E), lambda d, j: (j, 0)),
            pl.BlockSpec((1, E, H), lambda d, j: (d, 0, 0)),
            pl.BlockSpec((1, 1, H), lambda d, j: (d, 0, 0)),
        ],
        out_specs=pl.BlockSpec((1, VT, H), lambda d, j: (d, j, 0)),
        out_shape=jax.ShapeDtypeStruct((2, V, H), jnp.float32),
        compiler_params=pltpu.CompilerParams(
            dimension_semantics=("parallel", "arbitrary"),
        ),
        name="proj_table",
        interpret=interpret,
    )(emb, w_stack, bias)

    p2 = proj.reshape(2 * V * 2, 128)

    out = pl.pallas_call(
        _scan_kernel,
        grid_spec=pltpu.PrefetchScalarGridSpec(
            num_scalar_prefetch=1,
            grid=(2,),
            in_specs=[
                pl.BlockSpec(memory_space=pl.ANY),
                pl.BlockSpec((1, H, H), lambda d, ref: (d, 0, 0)),
                pl.BlockSpec((1, H, GPAD), lambda d, ref: (d, 0, 0)),
            ],
            out_specs=pl.BlockSpec((1, B, GPAD), lambda d, ref: (d, 0, 0)),
            scratch_shapes=[
                pltpu.VMEM((2 * V, 128), jnp.float32),
                pltpu.VMEM((2, 80, 128), jnp.float32),
                pltpu.SemaphoreType.DMA,
            ],
        ),
        out_shape=jax.ShapeDtypeStruct((2, B, GPAD), jnp.float32),
        compiler_params=pltpu.CompilerParams(
            dimension_semantics=("parallel",),
            vmem_limit_bytes=56 * 1024 * 1024,
        ),
        name="birnn_scan",
        interpret=interpret,
    )(idx2, p2, whh_stack, fcw_stack)

    res = out[0] + out[1] + jnp.pad(fc_b, (0, GPAD - G))
    return res[:, :G]


def kernel(token_ids, emb, W_ih_f, W_hh_f, b_ih_f, b_hh_f,
           W_ih_b, W_hh_b, b_ih_b, b_hh_b, fc_W, fc_b):
    return _impl(token_ids, emb, W_ih_f, W_hh_f, b_ih_f, b_hh_f,
                 W_ih_b, W_hh_b, b_ih_b, b_hh_b, fc_W, fc_b)


# speedup vs baseline: 17.5542x; 1.9011x over previous
import jax
import jax.numpy as jnp
from jax import lax
from jax.experimental import pallas as pl
from jax.experimental.pallas import tpu as pltpu

B, S, E, H, V, G = 32, 2048, 300, 256, 50000, 5
VT = 2000
N_VT = V // VT
GPAD = 128


def _bits(x):
    return lax.bitcast_convert_type(x, jnp.uint32)


def _pack(res):
    lo = res[:, :128].astype(jnp.bfloat16).astype(jnp.float32)
    hi = res[:, 128:].astype(jnp.bfloat16).astype(jnp.float32)
    packed = (_bits(lo) >> 16) | (_bits(hi) & jnp.uint32(0xFFFF0000))
    return lax.bitcast_convert_type(packed, jnp.int32)


def _proj_kernel(emb_ref, w_ref, b_ref, out_ref):
    emb = emb_ref[...]
    res_f = jnp.dot(emb, w_ref[0], preferred_element_type=jnp.float32) + b_ref[0]
    res_b = jnp.dot(emb, w_ref[1], preferred_element_type=jnp.float32) + b_ref[1]
    pf = _pack(res_f).reshape(VT, 1, 128)
    pb = _pack(res_b).reshape(VT, 1, 128)
    out_ref[...] = jnp.concatenate([pf, pb], axis=1)


def _unpack(r):
    lo = lax.bitcast_convert_type(r << 16, jnp.float32)
    hi = lax.bitcast_convert_type(r & jnp.int32(-65536), jnp.float32)
    return jnp.concatenate([lo, hi], axis=1)


def _scan_kernel(idx_ref, p_hbm, whh_ref, fcw_ref, fcb_ref, out_ref,
                 p_vmem, tf, tb, sem):
    cp = pltpu.make_async_copy(p_hbm, p_vmem, sem)
    cp.start()
    cp.wait()

    whh_f = whh_ref[0]
    whh_b = whh_ref[1]

    def gather(tstep_f, tstep_b, slot):
        base_f = tstep_f * B
        base_b = tstep_b * B
        for mi in range(B):
            i_f = pl.multiple_of(idx_ref[base_f + mi], 2)
            slab_f = p_vmem[pl.ds(i_f, 2), :]
            tf[slot, mi:mi + 1, :] = slab_f[0:1]
            i_b = pl.multiple_of(idx_ref[base_b + mi], 2)
            slab_b = p_vmem[pl.ds(i_b, 2), :]
            tb[slot, mi:mi + 1, :] = slab_b[1:2]

    gather(0, S - 1, 0)

    def step(t, carry):
        h_f, h_b, a_f, a_b = carry
        slot = lax.rem(t, 2)
        xp_f = _unpack(tf[slot])
        xp_b = _unpack(tb[slot])
        pre_f = jnp.dot(h_f, whh_f, preferred_element_type=jnp.float32) + xp_f
        pre_b = jnp.dot(h_b, whh_b, preferred_element_type=jnp.float32) + xp_b
        tn = jnp.clip(t + 1, 0, S - 1)
        gather(tn, (S - 1) - tn, 1 - slot)
        h_f2 = jnp.tanh(pre_f)
        h_b2 = jnp.tanh(pre_b)
        return h_f2, h_b2, a_f + h_f2, a_b + h_b2

    z = jnp.zeros((B, H), jnp.float32)
    _, _, a_f, a_b = lax.fori_loop(0, S, step, (z, z, z, z))

    act_f = jnp.tanh(a_f * (1.0 / S))
    act_b = jnp.tanh(a_b * (1.0 / S))
    out_ref[...] = (jnp.dot(act_f, fcw_ref[0], preferred_element_type=jnp.float32)
                    + jnp.dot(act_b, fcw_ref[1], preferred_element_type=jnp.float32)
                    + fcb_ref[...])


def _impl(token_ids, emb, W_ih_f, W_hh_f, b_ih_f, b_hh_f,
          W_ih_b, W_hh_b, b_ih_b, b_hh_b, fc_W, fc_b, interpret=False):
    w_stack = jnp.stack([W_ih_f.T, W_ih_b.T])
    bias = jnp.stack([(b_ih_f + b_hh_f), (b_ih_b + b_hh_b)])
    bias = bias.reshape(2, 1, H)
    whh_stack = jnp.stack([W_hh_f.T, W_hh_b.T])
    fcwT = fc_W.T
    fcw_stack = jnp.stack([fcwT[:H], fcwT[H:]])
    fcw_stack = jnp.pad(fcw_stack, ((0, 0), (0, 0), (0, GPAD - G)))
    fcb_pad = jnp.pad(fc_b, (0, GPAD - G)).reshape(1, GPAD)
    idx2 = (token_ids.T * 2).reshape(-1).astype(jnp.int32)

    packed = pl.pallas_call(
        _proj_kernel,
        grid=(N_VT,),
        in_specs=[
            pl.BlockSpec((VT, E), lambda j: (j, 0)),
            pl.BlockSpec((2, E, H), lambda j: (0, 0, 0)),
            pl.BlockSpec((2, 1, H), lambda j: (0, 0, 0)),
        ],
        out_specs=pl.BlockSpec((VT, 2, 128), lambda j: (j, 0, 0)),
        out_shape=jax.ShapeDtypeStruct((V, 2, 128), jnp.int32),
        compiler_params=pltpu.CompilerParams(
            dimension_semantics=("arbitrary",),
        ),
        name="proj_table",
        interpret=interpret,
    )(emb, w_stack, bias)

    p2 = packed.reshape(2 * V, 128)

    out = pl.pallas_call(
        _scan_kernel,
        grid_spec=pltpu.PrefetchScalarGridSpec(
            num_scalar_prefetch=1,
            grid=(1,),
            in_specs=[
                pl.BlockSpec(memory_space=pl.ANY),
                pl.BlockSpec((2, H, H), lambda i, ref: (0, 0, 0)),
                pl.BlockSpec((2, H, GPAD), lambda i, ref: (0, 0, 0)),
                pl.BlockSpec((1, GPAD), lambda i, ref: (0, 0)),
            ],
            out_specs=pl.BlockSpec((B, GPAD), lambda i, ref: (0, 0)),
            scratch_shapes=[
                pltpu.VMEM((2 * V, 128), jnp.int32),
                pltpu.VMEM((2, B, 128), jnp.int32),
                pltpu.VMEM((2, B, 128), jnp.int32),
                pltpu.SemaphoreType.DMA,
            ],
        ),
        out_shape=jax.ShapeDtypeStruct((B, GPAD), jnp.float32),
        compiler_params=pltpu.CompilerParams(
            dimension_semantics=("arbitrary",),
            vmem_limit_bytes=56 * 1024 * 1024,
        ),
        name="birnn_scan",
        interpret=interpret,
    )(idx2, p2, whh_stack, fcw_stack, fcb_pad)

    return out[:, :G]


def kernel(token_ids, emb, W_ih_f, W_hh_f, b_ih_f, b_hh_f,
           W_ih_b, W_hh_b, b_ih_b, b_hh_b, fc_W, fc_b):
    return _impl(token_ids, emb, W_ih_f, W_hh_f, b_ih_f, b_hh_f,
                 W_ih_b, W_hh_b, b_ih_b, b_hh_b, fc_W, fc_b)


# speedup vs baseline: 19.9763x; 1.1380x over previous
import jax
import jax.numpy as jnp
from jax import lax
from jax.experimental import pallas as pl
from jax.experimental.pallas import tpu as pltpu

B, S, E, H, V, G = 32, 2048, 300, 256, 50000, 5
VT = 2000
N_VT = V // VT
GPAD = 128


def _bits(x):
    return lax.bitcast_convert_type(x, jnp.uint32)


def _pack(res):
    lo = res[:, :128].astype(jnp.bfloat16).astype(jnp.float32)
    hi = res[:, 128:].astype(jnp.bfloat16).astype(jnp.float32)
    packed = (_bits(lo) >> 16) | (_bits(hi) & jnp.uint32(0xFFFF0000))
    return lax.bitcast_convert_type(packed, jnp.int32)


def _proj_kernel(emb_ref, w_ref, b_ref, out_ref):
    emb = emb_ref[...]
    res_f = jnp.dot(emb, w_ref[0], preferred_element_type=jnp.float32) + b_ref[0]
    res_b = jnp.dot(emb, w_ref[1], preferred_element_type=jnp.float32) + b_ref[1]
    pf = _pack(res_f).reshape(VT, 1, 128)
    pb = _pack(res_b).reshape(VT, 1, 128)
    out_ref[...] = jnp.concatenate([pf, pb], axis=1)


def _unpack(r):
    lo = lax.bitcast_convert_type(r << 16, jnp.float32)
    hi = lax.bitcast_convert_type(r & jnp.int32(-65536), jnp.float32)
    return jnp.concatenate([lo, hi], axis=1)


def _scan_kernel(idx_ref, p_hbm, whh_ref, fcw_ref, fcb_ref, out_ref,
                 p_vmem, tf, tb, sem):
    cp = pltpu.make_async_copy(p_hbm, p_vmem, sem)
    cp.start()
    cp.wait()

    def gather(tstep_f, tstep_b, slot):
        base_f = tstep_f * B
        base_b = tstep_b * B
        for mi in range(B):
            i_f = pl.multiple_of(idx_ref[base_f + mi], 2)
            slab_f = p_vmem[pl.ds(i_f, 2), :]
            tf[slot, mi:mi + 1, :] = slab_f[0:1]
            i_b = pl.multiple_of(idx_ref[base_b + mi], 2)
            slab_b = p_vmem[pl.ds(i_b, 2), :]
            tb[slot, mi:mi + 1, :] = slab_b[1:2]

    def mm_issue(h_f, h_b, lsr):
        pltpu.matmul_acc_lhs(0, h_f, mxu_index=0, load_staged_rhs=lsr)
        pltpu.matmul_acc_lhs(0, h_b, mxu_index=1, load_staged_rhs=lsr)

    def mm_pop():
        p_f = pltpu.matmul_pop(0, (B, H), jnp.float32, mxu_index=0)
        p_b = pltpu.matmul_pop(0, (B, H), jnp.float32, mxu_index=1)
        return p_f, p_b

    pltpu.matmul_push_rhs(whh_ref[0], 0, 0)
    pltpu.matmul_push_rhs(whh_ref[1], 0, 1)

    gather(0, S - 1, 0)
    gather(1, S - 2, 1)
    h_f = jnp.tanh(_unpack(tf[0]))
    h_b = jnp.tanh(_unpack(tb[0]))
    a_f = h_f
    a_b = h_b

    mm_issue(h_f, h_b, 0)
    xp_f = _unpack(tf[1])
    xp_b = _unpack(tb[1])
    gather(2, S - 3, 0)
    p_f, p_b = mm_pop()
    h_f = jnp.tanh(p_f + xp_f)
    h_b = jnp.tanh(p_b + xp_b)
    a_f = a_f + h_f
    a_b = a_b + h_b

    def step(t, carry):
        h_f, h_b, a_f, a_b = carry
        slot = lax.rem(t, 2)
        mm_issue(h_f, h_b, None)
        xp_f = _unpack(tf[slot])
        xp_b = _unpack(tb[slot])
        tn = jnp.clip(t + 1, 0, S - 1)
        gather(tn, (S - 1) - tn, 1 - slot)
        p_f, p_b = mm_pop()
        h_f2 = jnp.tanh(p_f + xp_f)
        h_b2 = jnp.tanh(p_b + xp_b)
        return h_f2, h_b2, a_f + h_f2, a_b + h_b2

    h_f, h_b, a_f, a_b = lax.fori_loop(2, S, step, (h_f, h_b, a_f, a_b))

    act_f = jnp.tanh(a_f * (1.0 / S))
    act_b = jnp.tanh(a_b * (1.0 / S))
    pltpu.matmul_push_rhs(fcw_ref[0], 0, 0)
    pltpu.matmul_push_rhs(fcw_ref[1], 0, 1)
    pltpu.matmul_acc_lhs(0, act_f, mxu_index=0, load_staged_rhs=0)
    pltpu.matmul_acc_lhs(0, act_b, mxu_index=1, load_staged_rhs=0)
    o_f = pltpu.matmul_pop(0, (B, H), jnp.float32, mxu_index=0)
    o_b = pltpu.matmul_pop(0, (B, H), jnp.float32, mxu_index=1)
    out_ref[...] = o_f[:, :GPAD] + o_b[:, :GPAD] + fcb_ref[...]


def _impl(token_ids, emb, W_ih_f, W_hh_f, b_ih_f, b_hh_f,
          W_ih_b, W_hh_b, b_ih_b, b_hh_b, fc_W, fc_b, interpret=False):
    w_stack = jnp.stack([W_ih_f.T, W_ih_b.T])
    bias = jnp.stack([(b_ih_f + b_hh_f), (b_ih_b + b_hh_b)])
    bias = bias.reshape(2, 1, H)
    whh_stack = jnp.stack([W_hh_f.T, W_hh_b.T])
    fcwT = fc_W.T
    fcw_stack = jnp.stack([fcwT[:H], fcwT[H:]])
    fcw_stack = jnp.pad(fcw_stack, ((0, 0), (0, 0), (0, H - G)))
    fcb_pad = jnp.pad(fc_b, (0, GPAD - G)).reshape(1, GPAD)
    idx2 = (token_ids.T * 2).reshape(-1).astype(jnp.int32)

    packed = pl.pallas_call(
        _proj_kernel,
        grid=(N_VT,),
        in_specs=[
            pl.BlockSpec((VT, E), lambda j: (j, 0)),
            pl.BlockSpec((2, E, H), lambda j: (0, 0, 0)),
            pl.BlockSpec((2, 1, H), lambda j: (0, 0, 0)),
        ],
        out_specs=pl.BlockSpec((VT, 2, 128), lambda j: (j, 0, 0)),
        out_shape=jax.ShapeDtypeStruct((V, 2, 128), jnp.int32),
        compiler_params=pltpu.CompilerParams(
            dimension_semantics=("arbitrary",),
        ),
        name="proj_table",
        interpret=interpret,
    )(emb, w_stack, bias)

    p2 = packed.reshape(2 * V, 128)

    out = pl.pallas_call(
        _scan_kernel,
        grid_spec=pltpu.PrefetchScalarGridSpec(
            num_scalar_prefetch=1,
            grid=(1,),
            in_specs=[
                pl.BlockSpec(memory_space=pl.ANY),
                pl.BlockSpec((2, H, H), lambda i, ref: (0, 0, 0)),
                pl.BlockSpec((2, H, H), lambda i, ref: (0, 0, 0)),
                pl.BlockSpec((1, GPAD), lambda i, ref: (0, 0)),
            ],
            out_specs=pl.BlockSpec((B, GPAD), lambda i, ref: (0, 0)),
            scratch_shapes=[
                pltpu.VMEM((2 * V, 128), jnp.int32),
                pltpu.VMEM((2, B, 128), jnp.int32),
                pltpu.VMEM((2, B, 128), jnp.int32),
                pltpu.SemaphoreType.DMA,
            ],
        ),
        out_shape=jax.ShapeDtypeStruct((B, GPAD), jnp.float32),
        compiler_params=pltpu.CompilerParams(
            dimension_semantics=("arbitrary",),
            vmem_limit_bytes=56 * 1024 * 1024,
        ),
        name="birnn_scan",
        interpret=interpret,
    )(idx2, p2, whh_stack, fcw_stack, fcb_pad)

    return out[:, :G]


def kernel(token_ids, emb, W_ih_f, W_hh_f, b_ih_f, b_hh_f,
           W_ih_b, W_hh_b, b_ih_b, b_hh_b, fc_W, fc_b):
    return _impl(token_ids, emb, W_ih_f, W_hh_f, b_ih_f, b_hh_f,
                 W_ih_b, W_hh_b, b_ih_b, b_hh_b, fc_W, fc_b)


# speedup vs baseline: 20.4292x; 1.0227x over previous
import jax
import jax.numpy as jnp
from jax import lax
from jax.experimental import pallas as pl
from jax.experimental.pallas import tpu as pltpu

B, S, E, H, V, G = 32, 2048, 300, 256, 50000, 5
VT = 2000
N_VT = V // VT
GPAD = 128


def _bits(x):
    return lax.bitcast_convert_type(x, jnp.uint32)


def _pack(res):
    lo = res[:, :128].astype(jnp.bfloat16).astype(jnp.float32)
    hi = res[:, 128:].astype(jnp.bfloat16).astype(jnp.float32)
    packed = (_bits(lo) >> 16) | (_bits(hi) & jnp.uint32(0xFFFF0000))
    return lax.bitcast_convert_type(packed, jnp.int32)


def _proj_kernel(emb_ref, w_ref, b_ref, out_ref):
    emb = emb_ref[...]
    res_f = jnp.dot(emb, w_ref[0], preferred_element_type=jnp.float32) + b_ref[0]
    res_b = jnp.dot(emb, w_ref[1], preferred_element_type=jnp.float32) + b_ref[1]
    pf = _pack(res_f).reshape(VT, 1, 128)
    pb = _pack(res_b).reshape(VT, 1, 128)
    out_ref[...] = jnp.concatenate([pf, pb], axis=1)


def _unpack(r):
    lo = lax.bitcast_convert_type(r << 16, jnp.float32)
    hi = lax.bitcast_convert_type(r & jnp.int32(-65536), jnp.float32)
    return jnp.concatenate([lo, hi], axis=1)


def _scan_kernel(idx_ref, p_hbm, whh_ref, fcw_ref, fcb_ref, out_ref,
                 p_vmem, tf, tb, af_ref, ab_ref, sem):
    cp = pltpu.make_async_copy(p_hbm, p_vmem, sem)
    cp.start()
    cp.wait()

    def gather(tstep_f, tstep_b, slot):
        base_f = tstep_f * B
        base_b = tstep_b * B
        for mi in range(B):
            i_f = pl.multiple_of(idx_ref[base_f + mi], 2)
            slab_f = p_vmem[pl.ds(i_f, 2), :]
            tf[slot, mi:mi + 1, :] = slab_f[0:1]
            i_b = pl.multiple_of(idx_ref[base_b + mi], 2)
            slab_b = p_vmem[pl.ds(i_b, 2), :]
            tb[slot, mi:mi + 1, :] = slab_b[1:2]

    def mm_issue(h_f, h_b, lsr):
        pltpu.matmul_acc_lhs(0, h_f, mxu_index=0, load_staged_rhs=lsr)
        pltpu.matmul_acc_lhs(0, h_b, mxu_index=1, load_staged_rhs=lsr)

    def mm_pop():
        p_f = pltpu.matmul_pop(0, (B, H), jnp.float32, mxu_index=0)
        p_b = pltpu.matmul_pop(0, (B, H), jnp.float32, mxu_index=1)
        return p_f, p_b

    pltpu.matmul_push_rhs(whh_ref[0], 0, 0)
    pltpu.matmul_push_rhs(whh_ref[1], 0, 1)

    gather(0, S - 1, 0)
    gather(1, S - 2, 1)
    h_f = jnp.tanh(_unpack(tf[0]))
    h_b = jnp.tanh(_unpack(tb[0]))
    af_ref[...] = h_f
    ab_ref[...] = h_b

    mm_issue(h_f, h_b, 0)
    xp_f = _unpack(tf[1])
    xp_b = _unpack(tb[1])
    gather(2, S - 3, 0)
    gather(3, S - 4, 1)
    p_f, p_b = mm_pop()
    h_f = jnp.tanh(p_f + xp_f)
    h_b = jnp.tanh(p_b + xp_b)
    af_ref[...] = af_ref[...] + h_f
    ab_ref[...] = ab_ref[...] + h_b

    def substep(t, h_f, h_b, slot, dist):
        mm_issue(h_f, h_b, None)
        xp_f = _unpack(tf[slot])
        xp_b = _unpack(tb[slot])
        tn = jnp.clip(t + dist, 0, S - 1)
        gather(tn, (S - 1) - tn, slot)
        p_f, p_b = mm_pop()
        h_f2 = jnp.tanh(p_f + xp_f)
        h_b2 = jnp.tanh(p_b + xp_b)
        af_ref[...] = af_ref[...] + h_f2
        ab_ref[...] = ab_ref[...] + h_b2
        return h_f2, h_b2

    def step(k, carry):
        h_f, h_b = carry
        t = 2 + 2 * k
        h_f, h_b = substep(t, h_f, h_b, 0, 2)
        h_f, h_b = substep(t + 1, h_f, h_b, 1, 2)
        return h_f, h_b

    lax.fori_loop(0, (S - 2) // 2, step, (h_f, h_b))

    act_f = jnp.tanh(af_ref[...] * (1.0 / S))
    act_b = jnp.tanh(ab_ref[...] * (1.0 / S))
    pltpu.matmul_push_rhs(fcw_ref[0], 0, 0)
    pltpu.matmul_push_rhs(fcw_ref[1], 0, 1)
    pltpu.matmul_acc_lhs(0, act_f, mxu_index=0, load_staged_rhs=0)
    pltpu.matmul_acc_lhs(0, act_b, mxu_index=1, load_staged_rhs=0)
    o_f = pltpu.matmul_pop(0, (B, H), jnp.float32, mxu_index=0)
    o_b = pltpu.matmul_pop(0, (B, H), jnp.float32, mxu_index=1)
    out_ref[...] = o_f[:, :GPAD] + o_b[:, :GPAD] + fcb_ref[...]


def _impl(token_ids, emb, W_ih_f, W_hh_f, b_ih_f, b_hh_f,
          W_ih_b, W_hh_b, b_ih_b, b_hh_b, fc_W, fc_b, interpret=False):
    w_stack = jnp.stack([W_ih_f.T, W_ih_b.T]).astype(jnp.bfloat16)
    emb_bf = emb.astype(jnp.bfloat16)
    bias = jnp.stack([(b_ih_f + b_hh_f), (b_ih_b + b_hh_b)])
    bias = bias.reshape(2, 1, H)
    whh_stack = jnp.stack([W_hh_f.T, W_hh_b.T])
    fcwT = fc_W.T
    fcw_stack = jnp.stack([fcwT[:H], fcwT[H:]])
    fcw_stack = jnp.pad(fcw_stack, ((0, 0), (0, 0), (0, H - G)))
    fcb_pad = jnp.pad(fc_b, (0, GPAD - G)).reshape(1, GPAD)
    idx2 = (token_ids.T * 2).reshape(-1).astype(jnp.int32)

    packed = pl.pallas_call(
        _proj_kernel,
        grid=(N_VT,),
        in_specs=[
            pl.BlockSpec((VT, E), lambda j: (j, 0)),
            pl.BlockSpec((2, E, H), lambda j: (0, 0, 0)),
            pl.BlockSpec((2, 1, H), lambda j: (0, 0, 0)),
        ],
        out_specs=pl.BlockSpec((VT, 2, 128), lambda j: (j, 0, 0)),
        out_shape=jax.ShapeDtypeStruct((V, 2, 128), jnp.int32),
        compiler_params=pltpu.CompilerParams(
            dimension_semantics=("arbitrary",),
        ),
        name="proj_table",
        interpret=interpret,
    )(emb_bf, w_stack, bias)

    p2 = packed.reshape(2 * V, 128)

    out = pl.pallas_call(
        _scan_kernel,
        grid_spec=pltpu.PrefetchScalarGridSpec(
            num_scalar_prefetch=1,
            grid=(1,),
            in_specs=[
                pl.BlockSpec(memory_space=pl.ANY),
                pl.BlockSpec((2, H, H), lambda i, ref: (0, 0, 0)),
                pl.BlockSpec((2, H, H), lambda i, ref: (0, 0, 0)),
                pl.BlockSpec((1, GPAD), lambda i, ref: (0, 0)),
            ],
            out_specs=pl.BlockSpec((B, GPAD), lambda i, ref: (0, 0)),
            scratch_shapes=[
                pltpu.VMEM((2 * V, 128), jnp.int32),
                pltpu.VMEM((2, B, 128), jnp.int32),
                pltpu.VMEM((2, B, 128), jnp.int32),
                pltpu.VMEM((B, H), jnp.float32),
                pltpu.VMEM((B, H), jnp.float32),
                pltpu.SemaphoreType.DMA,
            ],
        ),
        out_shape=jax.ShapeDtypeStruct((B, GPAD), jnp.float32),
        compiler_params=pltpu.CompilerParams(
            dimension_semantics=("arbitrary",),
            vmem_limit_bytes=56 * 1024 * 1024,
        ),
        name="birnn_scan",
        interpret=interpret,
    )(idx2, p2, whh_stack, fcw_stack, fcb_pad)

    return out[:, :G]


def kernel(token_ids, emb, W_ih_f, W_hh_f, b_ih_f, b_hh_f,
           W_ih_b, W_hh_b, b_ih_b, b_hh_b, fc_W, fc_b):
    return _impl(token_ids, emb, W_ih_f, W_hh_f, b_ih_f, b_hh_f,
                 W_ih_b, W_hh_b, b_ih_b, b_hh_b, fc_W, fc_b)


# speedup vs baseline: 21.8385x; 1.0690x over previous
import jax
import jax.numpy as jnp
from jax import lax
from jax.experimental import pallas as pl
from jax.experimental.pallas import tpu as pltpu

B, S, E, H, V, G = 32, 2048, 300, 256, 50000, 5
VT = 2000
N_VT = V // VT
GPAD = 128


def _bits(x):
    return lax.bitcast_convert_type(x, jnp.uint32)


def _pack(res):
    lo = res[:, :128].astype(jnp.bfloat16).astype(jnp.float32)
    hi = res[:, 128:].astype(jnp.bfloat16).astype(jnp.float32)
    packed = (_bits(lo) >> 16) | (_bits(hi) & jnp.uint32(0xFFFF0000))
    return lax.bitcast_convert_type(packed, jnp.int32)


def _proj_kernel(emb_ref, w_ref, b_ref, out_ref):
    emb = emb_ref[...].astype(jnp.bfloat16)
    res_f = jnp.dot(emb, w_ref[0], preferred_element_type=jnp.float32) + b_ref[0]
    res_b = jnp.dot(emb, w_ref[1], preferred_element_type=jnp.float32) + b_ref[1]
    out_ref[0:2 * VT:2, :] = _pack(res_f)
    out_ref[1:2 * VT:2, :] = _pack(res_b)


def _unpack(r):
    lo = lax.bitcast_convert_type(r << 16, jnp.float32)
    hi = lax.bitcast_convert_type(r & jnp.int32(-65536), jnp.float32)
    return jnp.concatenate([lo, hi], axis=1)


def _scan_kernel(idx_ref, p_hbm, whh_ref, fcw_ref, fcb_ref, out_ref,
                 p_vmem, tf, tb, af_ref, ab_ref, sem):
    cp = pltpu.make_async_copy(p_hbm, p_vmem, sem)
    cp.start()
    cp.wait()

    def gather(tstep_f, tstep_b, slot):
        for mi in range(B):
            i_f = pl.multiple_of(idx_ref[mi * S + tstep_f], 2)
            slab_f = p_vmem[pl.ds(i_f, 2), :]
            tf[slot, mi:mi + 1, :] = slab_f[0:1]
            i_b = pl.multiple_of(idx_ref[mi * S + tstep_b], 2)
            slab_b = p_vmem[pl.ds(i_b, 2), :]
            tb[slot, mi:mi + 1, :] = slab_b[1:2]

    def mm_issue(h_f, h_b, lsr):
        pltpu.matmul_acc_lhs(0, h_f, mxu_index=0, load_staged_rhs=lsr)
        pltpu.matmul_acc_lhs(0, h_b, mxu_index=1, load_staged_rhs=lsr)

    def mm_pop():
        p_f = pltpu.matmul_pop(0, (B, H), jnp.float32, mxu_index=0)
        p_b = pltpu.matmul_pop(0, (B, H), jnp.float32, mxu_index=1)
        return p_f, p_b

    pltpu.matmul_push_rhs(whh_ref[0], 0, 0)
    pltpu.matmul_push_rhs(whh_ref[1], 0, 1)

    gather(0, S - 1, 0)
    gather(1, S - 2, 1)
    h_f = jnp.tanh(_unpack(tf[0]))
    h_b = jnp.tanh(_unpack(tb[0]))
    af_ref[...] = h_f
    ab_ref[...] = h_b

    mm_issue(h_f, h_b, 0)
    xp_f = _unpack(tf[1])
    xp_b = _unpack(tb[1])
    gather(2, S - 3, 0)
    gather(3, S - 4, 1)
    p_f, p_b = mm_pop()
    h_f = jnp.tanh(p_f + xp_f)
    h_b = jnp.tanh(p_b + xp_b)
    af_ref[...] = af_ref[...] + h_f
    ab_ref[...] = ab_ref[...] + h_b

    def substep(t, h_f, h_b, slot, dist):
        mm_issue(h_f, h_b, None)
        xp_f = _unpack(tf[slot])
        xp_b = _unpack(tb[slot])
        tn = jnp.clip(t + dist, 0, S - 1)
        gather(tn, (S - 1) - tn, slot)
        p_f, p_b = mm_pop()
        h_f2 = jnp.tanh(p_f + xp_f)
        h_b2 = jnp.tanh(p_b + xp_b)
        af_ref[...] = af_ref[...] + h_f2
        ab_ref[...] = ab_ref[...] + h_b2
        return h_f2, h_b2

    def step(k, carry):
        h_f, h_b = carry
        t = 2 + 2 * k
        h_f, h_b = substep(t, h_f, h_b, 0, 2)
        h_f, h_b = substep(t + 1, h_f, h_b, 1, 2)
        return h_f, h_b

    lax.fori_loop(0, (S - 2) // 2, step, (h_f, h_b))

    act_f = jnp.tanh(af_ref[...] * (1.0 / S))
    act_b = jnp.tanh(ab_ref[...] * (1.0 / S))
    pltpu.matmul_push_rhs(fcw_ref[0], 0, 0)
    pltpu.matmul_push_rhs(fcw_ref[1], 0, 1)
    pltpu.matmul_acc_lhs(0, act_f, mxu_index=0, load_staged_rhs=0)
    pltpu.matmul_acc_lhs(0, act_b, mxu_index=1, load_staged_rhs=0)
    o_f = pltpu.matmul_pop(0, (B, H), jnp.float32, mxu_index=0)
    o_b = pltpu.matmul_pop(0, (B, H), jnp.float32, mxu_index=1)
    out_ref[...] = o_f[:, :GPAD] + o_b[:, :GPAD] + fcb_ref[...]


def _impl(token_ids, emb, W_ih_f, W_hh_f, b_ih_f, b_hh_f,
          W_ih_b, W_hh_b, b_ih_b, b_hh_b, fc_W, fc_b, interpret=False):
    w_stack = jnp.stack([W_ih_f.T, W_ih_b.T]).astype(jnp.bfloat16)
    bias = jnp.stack([(b_ih_f + b_hh_f), (b_ih_b + b_hh_b)])
    bias = bias.reshape(2, 1, H)
    whh_stack = jnp.stack([W_hh_f.T, W_hh_b.T])
    fcwT = fc_W.T
    fcw_stack = jnp.stack([fcwT[:H], fcwT[H:]])
    fcw_stack = jnp.pad(fcw_stack, ((0, 0), (0, 0), (0, H - G)))
    fcb_pad = jnp.pad(fc_b, (0, GPAD - G)).reshape(1, GPAD)
    idx2 = (token_ids * 2).reshape(-1).astype(jnp.int32)

    packed = pl.pallas_call(
        _proj_kernel,
        grid=(N_VT,),
        in_specs=[
            pl.BlockSpec((VT, E), lambda j: (j, 0)),
            pl.BlockSpec((2, E, H), lambda j: (0, 0, 0)),
            pl.BlockSpec((2, 1, H), lambda j: (0, 0, 0)),
        ],
        out_specs=pl.BlockSpec((2 * VT, 128), lambda j: (j, 0)),
        out_shape=jax.ShapeDtypeStruct((2 * V, 128), jnp.int32),
        compiler_params=pltpu.CompilerParams(
            dimension_semantics=("arbitrary",),
        ),
        name="proj_table",
        interpret=interpret,
    )(emb, w_stack, bias)

    p2 = packed

    out = pl.pallas_call(
        _scan_kernel,
        grid_spec=pltpu.PrefetchScalarGridSpec(
            num_scalar_prefetch=1,
            grid=(1,),
            in_specs=[
                pl.BlockSpec(memory_space=pl.ANY),
                pl.BlockSpec((2, H, H), lambda i, ref: (0, 0, 0)),
                pl.BlockSpec((2, H, H), lambda i, ref: (0, 0, 0)),
                pl.BlockSpec((1, GPAD), lambda i, ref: (0, 0)),
            ],
            out_specs=pl.BlockSpec((B, GPAD), lambda i, ref: (0, 0)),
            scratch_shapes=[
                pltpu.VMEM((2 * V, 128), jnp.int32),
                pltpu.VMEM((2, B, 128), jnp.int32),
                pltpu.VMEM((2, B, 128), jnp.int32),
                pltpu.VMEM((B, H), jnp.float32),
                pltpu.VMEM((B, H), jnp.float32),
                pltpu.SemaphoreType.DMA,
            ],
        ),
        out_shape=jax.ShapeDtypeStruct((B, GPAD), jnp.float32),
        compiler_params=pltpu.CompilerParams(
            dimension_semantics=("arbitrary",),
            vmem_limit_bytes=56 * 1024 * 1024,
        ),
        name="birnn_scan",
        interpret=interpret,
    )(idx2, p2, whh_stack, fcw_stack, fcb_pad)

    return out[:, :G]


def kernel(token_ids, emb, W_ih_f, W_hh_f, b_ih_f, b_hh_f,
           W_ih_b, W_hh_b, b_ih_b, b_hh_b, fc_W, fc_b):
    return _impl(token_ids, emb, W_ih_f, W_hh_f, b_ih_f, b_hh_f,
                 W_ih_b, W_hh_b, b_ih_b, b_hh_b, fc_W, fc_b)


# speedup vs baseline: 22.2543x; 1.0190x over previous
import jax
import jax.numpy as jnp
from jax import lax
from jax.experimental import pallas as pl
from jax.experimental.pallas import tpu as pltpu

B, S, E, H, V, G = 32, 2048, 300, 256, 50000, 5
VT = 5000
N_VT = V // VT
GPAD = 128


def _bits(x):
    return lax.bitcast_convert_type(x, jnp.uint32)


def _pack(res):
    lo = res[:, :128].astype(jnp.bfloat16).astype(jnp.float32)
    hi = res[:, 128:].astype(jnp.bfloat16).astype(jnp.float32)
    packed = (_bits(lo) >> 16) | (_bits(hi) & jnp.uint32(0xFFFF0000))
    return lax.bitcast_convert_type(packed, jnp.int32)


def _proj_kernel(emb_ref, w_ref, b_ref, out_ref):
    emb = emb_ref[...].astype(jnp.bfloat16)
    res_f = jnp.dot(emb, w_ref[0], preferred_element_type=jnp.float32) + b_ref[0]
    res_b = jnp.dot(emb, w_ref[1], preferred_element_type=jnp.float32) + b_ref[1]
    out_ref[0:2 * VT:2, :] = _pack(res_f)
    out_ref[1:2 * VT:2, :] = _pack(res_b)


def _unpack(r):
    lo = lax.bitcast_convert_type(r << 16, jnp.float32)
    hi = lax.bitcast_convert_type(r & jnp.int32(-65536), jnp.float32)
    return jnp.concatenate([lo, hi], axis=1)


def _scan_kernel(idx_ref, p_hbm, whh_ref, fcw_ref, fcb_ref, out_ref,
                 p_vmem, tf, tb, af_ref, ab_ref, sem):
    cp = pltpu.make_async_copy(p_hbm, p_vmem, sem)
    cp.start()
    cp.wait()

    def gather(tstep_f, tstep_b, slot):
        for mi in range(B):
            i_f = pl.multiple_of(idx_ref[mi * S + tstep_f], 2)
            slab_f = p_vmem[pl.ds(i_f, 2), :]
            tf[slot, mi:mi + 1, :] = slab_f[0:1]
            i_b = pl.multiple_of(idx_ref[mi * S + tstep_b], 2)
            slab_b = p_vmem[pl.ds(i_b, 2), :]
            tb[slot, mi:mi + 1, :] = slab_b[1:2]

    def mm_issue(h_f, h_b, lsr):
        pltpu.matmul_acc_lhs(0, h_f, mxu_index=0, load_staged_rhs=lsr)
        pltpu.matmul_acc_lhs(0, h_b, mxu_index=1, load_staged_rhs=lsr)

    def mm_pop():
        p_f = pltpu.matmul_pop(0, (B, H), jnp.float32, mxu_index=0)
        p_b = pltpu.matmul_pop(0, (B, H), jnp.float32, mxu_index=1)
        return p_f, p_b

    pltpu.matmul_push_rhs(whh_ref[0], 0, 0)
    pltpu.matmul_push_rhs(whh_ref[1], 0, 1)

    gather(0, S - 1, 0)
    gather(1, S - 2, 1)
    h_f = jnp.tanh(_unpack(tf[0]))
    h_b = jnp.tanh(_unpack(tb[0]))
    af_ref[...] = h_f
    ab_ref[...] = h_b

    mm_issue(h_f, h_b, 0)
    xp_f = _unpack(tf[1])
    xp_b = _unpack(tb[1])
    gather(2, S - 3, 0)
    gather(3, S - 4, 1)
    p_f, p_b = mm_pop()
    h_f = jnp.tanh(p_f + xp_f)
    h_b = jnp.tanh(p_b + xp_b)
    af_ref[...] = af_ref[...] + h_f
    ab_ref[...] = ab_ref[...] + h_b

    def substep(t, h_f, h_b, slot, dist):
        mm_issue(h_f, h_b, None)
        xp_f = _unpack(tf[slot])
        xp_b = _unpack(tb[slot])
        tn = jnp.clip(t + dist, 0, S - 1)
        gather(tn, (S - 1) - tn, slot)
        p_f, p_b = mm_pop()
        h_f2 = jnp.tanh(p_f + xp_f)
        h_b2 = jnp.tanh(p_b + xp_b)
        af_ref[...] = af_ref[...] + h_f2
        ab_ref[...] = ab_ref[...] + h_b2
        return h_f2, h_b2

    def step(k, carry):
        h_f, h_b = carry
        t = 2 + 2 * k
        h_f, h_b = substep(t, h_f, h_b, 0, 2)
        h_f, h_b = substep(t + 1, h_f, h_b, 1, 2)
        return h_f, h_b

    lax.fori_loop(0, (S - 2) // 2, step, (h_f, h_b))

    act_f = jnp.tanh(af_ref[...] * (1.0 / S))
    act_b = jnp.tanh(ab_ref[...] * (1.0 / S))
    pltpu.matmul_push_rhs(fcw_ref[0], 0, 0)
    pltpu.matmul_push_rhs(fcw_ref[1], 0, 1)
    pltpu.matmul_acc_lhs(0, act_f, mxu_index=0, load_staged_rhs=0)
    pltpu.matmul_acc_lhs(0, act_b, mxu_index=1, load_staged_rhs=0)
    o_f = pltpu.matmul_pop(0, (B, H), jnp.float32, mxu_index=0)
    o_b = pltpu.matmul_pop(0, (B, H), jnp.float32, mxu_index=1)
    out_ref[...] = o_f[:, :GPAD] + o_b[:, :GPAD] + fcb_ref[...]


def _impl(token_ids, emb, W_ih_f, W_hh_f, b_ih_f, b_hh_f,
          W_ih_b, W_hh_b, b_ih_b, b_hh_b, fc_W, fc_b, interpret=False):
    w_stack = jnp.stack([W_ih_f.T, W_ih_b.T]).astype(jnp.bfloat16)
    bias = jnp.stack([(b_ih_f + b_hh_f), (b_ih_b + b_hh_b)])
    bias = bias.reshape(2, 1, H)
    whh_stack = jnp.stack([W_hh_f.T, W_hh_b.T])
    fcwT = fc_W.T
    fcw_stack = jnp.stack([fcwT[:H], fcwT[H:]])
    fcw_stack = jnp.pad(fcw_stack, ((0, 0), (0, 0), (0, H - G)))
    fcb_pad = jnp.pad(fc_b, (0, GPAD - G)).reshape(1, GPAD)
    idx2 = (token_ids * 2).reshape(-1).astype(jnp.int32)

    packed = pl.pallas_call(
        _proj_kernel,
        grid=(N_VT,),
        in_specs=[
            pl.BlockSpec((VT, E), lambda j: (j, 0)),
            pl.BlockSpec((2, E, H), lambda j: (0, 0, 0)),
            pl.BlockSpec((2, 1, H), lambda j: (0, 0, 0)),
        ],
        out_specs=pl.BlockSpec((2 * VT, 128), lambda j: (j, 0)),
        out_shape=jax.ShapeDtypeStruct((2 * V, 128), jnp.int32),
        compiler_params=pltpu.CompilerParams(
            dimension_semantics=("arbitrary",),
        ),
        name="proj_table",
        interpret=interpret,
    )(emb, w_stack, bias)

    p2 = packed

    out = pl.pallas_call(
        _scan_kernel,
        grid_spec=pltpu.PrefetchScalarGridSpec(
            num_scalar_prefetch=1,
            grid=(1,),
            in_specs=[
                pl.BlockSpec(memory_space=pl.ANY),
                pl.BlockSpec((2, H, H), lambda i, ref: (0, 0, 0)),
                pl.BlockSpec((2, H, H), lambda i, ref: (0, 0, 0)),
                pl.BlockSpec((1, GPAD), lambda i, ref: (0, 0)),
            ],
            out_specs=pl.BlockSpec((B, GPAD), lambda i, ref: (0, 0)),
            scratch_shapes=[
                pltpu.VMEM((2 * V, 128), jnp.int32),
                pltpu.VMEM((2, B, 128), jnp.int32),
                pltpu.VMEM((2, B, 128), jnp.int32),
                pltpu.VMEM((B, H), jnp.float32),
                pltpu.VMEM((B, H), jnp.float32),
                pltpu.SemaphoreType.DMA,
            ],
        ),
        out_shape=jax.ShapeDtypeStruct((B, GPAD), jnp.float32),
        compiler_params=pltpu.CompilerParams(
            dimension_semantics=("arbitrary",),
            vmem_limit_bytes=56 * 1024 * 1024,
        ),
        name="birnn_scan",
        interpret=interpret,
    )(idx2, p2, whh_stack, fcw_stack, fcb_pad)

    return out[:, :G]


def kernel(token_ids, emb, W_ih_f, W_hh_f, b_ih_f, b_hh_f,
           W_ih_b, W_hh_b, b_ih_b, b_hh_b, fc_W, fc_b):
    return _impl(token_ids, emb, W_ih_f, W_hh_f, b_ih_f, b_hh_f,
                 W_ih_b, W_hh_b, b_ih_b, b_hh_b, fc_W, fc_b)


# speedup vs baseline: 22.2703x; 1.0007x over previous
import jax
import jax.numpy as jnp
from jax import lax
from jax.experimental import pallas as pl
from jax.experimental.pallas import tpu as pltpu

B, S, E, H, V, G = 32, 2048, 300, 256, 50000, 5
VT = 5000
N_VT = V // VT
GPAD = 128


def _bits(x):
    return lax.bitcast_convert_type(x, jnp.uint32)


def _pack(res):
    lo = res[:, :128].astype(jnp.bfloat16).astype(jnp.float32)
    hi = res[:, 128:].astype(jnp.bfloat16).astype(jnp.float32)
    packed = (_bits(lo) >> 16) | (_bits(hi) & jnp.uint32(0xFFFF0000))
    return lax.bitcast_convert_type(packed, jnp.int32)


def _proj_kernel(emb_ref, w_ref, b_ref, out_ref):
    emb = emb_ref[...].astype(jnp.bfloat16)
    res_f = jnp.dot(emb, w_ref[0], preferred_element_type=jnp.float32) + b_ref[0]
    res_b = jnp.dot(emb, w_ref[1], preferred_element_type=jnp.float32) + b_ref[1]
    out_ref[0:2 * VT:2, :] = _pack(res_f)
    out_ref[1:2 * VT:2, :] = _pack(res_b)


def _unpack(r):
    lo = lax.bitcast_convert_type(r << 16, jnp.float32)
    hi = lax.bitcast_convert_type(r & jnp.int32(-65536), jnp.float32)
    return jnp.concatenate([lo, hi], axis=1)


def _scan_kernel(idx_ref, p_hbm, whh_ref, fcw_ref, fcb_ref, out_ref,
                 p_vmem, tf, tb, af_ref, ab_ref, sem):
    nsplit = 4
    rows = 2 * V // nsplit
    cps = [pltpu.make_async_copy(p_hbm.at[pl.ds(k * rows, rows), :],
                                 p_vmem.at[pl.ds(k * rows, rows), :],
                                 sem.at[k]) for k in range(nsplit)]
    for cp in cps:
        cp.start()
    pltpu.matmul_push_rhs(whh_ref[0], 0, 0)
    pltpu.matmul_push_rhs(whh_ref[1], 0, 1)
    for cp in cps:
        cp.wait()

    def gather(tstep_f, tstep_b, slot):
        for mi in range(B):
            i_f = pl.multiple_of(idx_ref[mi * S + tstep_f], 2)
            slab_f = p_vmem[pl.ds(i_f, 2), :]
            tf[slot, mi:mi + 1, :] = slab_f[0:1]
            i_b = pl.multiple_of(idx_ref[mi * S + tstep_b], 2)
            slab_b = p_vmem[pl.ds(i_b, 2), :]
            tb[slot, mi:mi + 1, :] = slab_b[1:2]

    def mm_issue(h_f, h_b, lsr):
        pltpu.matmul_acc_lhs(0, h_f, mxu_index=0, load_staged_rhs=lsr)
        pltpu.matmul_acc_lhs(0, h_b, mxu_index=1, load_staged_rhs=lsr)

    def mm_pop():
        p_f = pltpu.matmul_pop(0, (B, H), jnp.float32, mxu_index=0)
        p_b = pltpu.matmul_pop(0, (B, H), jnp.float32, mxu_index=1)
        return p_f, p_b

    gather(0, S - 1, 0)
    gather(1, S - 2, 1)
    h_f = jnp.tanh(_unpack(tf[0]))
    h_b = jnp.tanh(_unpack(tb[0]))
    af_ref[...] = h_f
    ab_ref[...] = h_b

    mm_issue(h_f, h_b, 0)
    xp_f = _unpack(tf[1])
    xp_b = _unpack(tb[1])
    gather(2, S - 3, 0)
    gather(3, S - 4, 1)
    p_f, p_b = mm_pop()
    h_f = jnp.tanh(p_f + xp_f)
    h_b = jnp.tanh(p_b + xp_b)
    af_ref[...] = af_ref[...] + h_f
    ab_ref[...] = ab_ref[...] + h_b

    def substep(t, h_f, h_b, slot, dist):
        mm_issue(h_f, h_b, None)
        xp_f = _unpack(tf[slot])
        xp_b = _unpack(tb[slot])
        tn = jnp.clip(t + dist, 0, S - 1)
        gather(tn, (S - 1) - tn, slot)
        p_f, p_b = mm_pop()
        h_f2 = jnp.tanh(p_f + xp_f)
        h_b2 = jnp.tanh(p_b + xp_b)
        af_ref[...] = af_ref[...] + h_f2
        ab_ref[...] = ab_ref[...] + h_b2
        return h_f2, h_b2

    def step(k, carry):
        h_f, h_b = carry
        t = 2 + 2 * k
        h_f, h_b = substep(t, h_f, h_b, 0, 2)
        h_f, h_b = substep(t + 1, h_f, h_b, 1, 2)
        return h_f, h_b

    lax.fori_loop(0, (S - 2) // 2, step, (h_f, h_b))

    act_f = jnp.tanh(af_ref[...] * (1.0 / S))
    act_b = jnp.tanh(ab_ref[...] * (1.0 / S))
    pltpu.matmul_push_rhs(fcw_ref[0], 0, 0)
    pltpu.matmul_push_rhs(fcw_ref[1], 0, 1)
    pltpu.matmul_acc_lhs(0, act_f, mxu_index=0, load_staged_rhs=0)
    pltpu.matmul_acc_lhs(0, act_b, mxu_index=1, load_staged_rhs=0)
    o_f = pltpu.matmul_pop(0, (B, H), jnp.float32, mxu_index=0)
    o_b = pltpu.matmul_pop(0, (B, H), jnp.float32, mxu_index=1)
    out_ref[...] = o_f[:, :GPAD] + o_b[:, :GPAD] + fcb_ref[...]


def _impl(token_ids, emb, W_ih_f, W_hh_f, b_ih_f, b_hh_f,
          W_ih_b, W_hh_b, b_ih_b, b_hh_b, fc_W, fc_b, interpret=False):
    w_stack = jnp.stack([W_ih_f.T, W_ih_b.T]).astype(jnp.bfloat16)
    bias = jnp.stack([(b_ih_f + b_hh_f), (b_ih_b + b_hh_b)])
    bias = bias.reshape(2, 1, H)
    whh_stack = jnp.stack([W_hh_f.T, W_hh_b.T])
    fcwT = fc_W.T
    fcw_stack = jnp.stack([fcwT[:H], fcwT[H:]])
    fcw_stack = jnp.pad(fcw_stack, ((0, 0), (0, 0), (0, H - G)))
    fcb_pad = jnp.pad(fc_b, (0, GPAD - G)).reshape(1, GPAD)
    idx2 = (token_ids * 2).reshape(-1).astype(jnp.int32)

    packed = pl.pallas_call(
        _proj_kernel,
        grid=(N_VT,),
        in_specs=[
            pl.BlockSpec((VT, E), lambda j: (j, 0)),
            pl.BlockSpec((2, E, H), lambda j: (0, 0, 0)),
            pl.BlockSpec((2, 1, H), lambda j: (0, 0, 0)),
        ],
        out_specs=pl.BlockSpec((2 * VT, 128), lambda j: (j, 0)),
        out_shape=jax.ShapeDtypeStruct((2 * V, 128), jnp.int32),
        compiler_params=pltpu.CompilerParams(
            dimension_semantics=("arbitrary",),
        ),
        name="proj_table",
        interpret=interpret,
    )(emb, w_stack, bias)

    p2 = packed

    out = pl.pallas_call(
        _scan_kernel,
        grid_spec=pltpu.PrefetchScalarGridSpec(
            num_scalar_prefetch=1,
            grid=(1,),
            in_specs=[
                pl.BlockSpec(memory_space=pl.ANY),
                pl.BlockSpec((2, H, H), lambda i, ref: (0, 0, 0)),
                pl.BlockSpec((2, H, H), lambda i, ref: (0, 0, 0)),
                pl.BlockSpec((1, GPAD), lambda i, ref: (0, 0)),
            ],
            out_specs=pl.BlockSpec((B, GPAD), lambda i, ref: (0, 0)),
            scratch_shapes=[
                pltpu.VMEM((2 * V, 128), jnp.int32),
                pltpu.VMEM((2, B, 128), jnp.int32),
                pltpu.VMEM((2, B, 128), jnp.int32),
                pltpu.VMEM((B, H), jnp.float32),
                pltpu.VMEM((B, H), jnp.float32),
                pltpu.SemaphoreType.DMA((4,)),
            ],
        ),
        out_shape=jax.ShapeDtypeStruct((B, GPAD), jnp.float32),
        compiler_params=pltpu.CompilerParams(
            dimension_semantics=("arbitrary",),
            vmem_limit_bytes=56 * 1024 * 1024,
        ),
        name="birnn_scan",
        interpret=interpret,
    )(idx2, p2, whh_stack, fcw_stack, fcb_pad)

    return out[:, :G]


def kernel(token_ids, emb, W_ih_f, W_hh_f, b_ih_f, b_hh_f,
           W_ih_b, W_hh_b, b_ih_b, b_hh_b, fc_W, fc_b):
    return _impl(token_ids, emb, W_ih_f, W_hh_f, b_ih_f, b_hh_f,
                 W_ih_b, W_hh_b, b_ih_b, b_hh_b, fc_W, fc_b)
